# baseline (speedup 1.0000x reference)
import functools

import jax
import jax.numpy as jnp
import numpy as np
from jax import lax
from jax.experimental import pallas as pl
from jax.experimental.pallas import tpu as pltpu
from jax.experimental.pallas import tpu_sc as plsc

STEP = 200
DIM = 16
BATCH = 1024

NC = 2
NS = 16
NW = NC * NS

B_FLAT = BATCH * STEP
B_PER_W = B_FLAT // NW
CHUNK = 128
N_CHUNKS = B_PER_W // CHUNK
SEQ_PER_W = B_PER_W // STEP


def _pe_table() -> np.ndarray:
    pos = np.arange(STEP)[:, None]
    with np.errstate(divide="ignore", invalid="ignore", over="ignore"):
        pe = pos / (np.power(1000, 2 * np.arange(DIM, dtype=np.int64))[None, :] / DIM)
        pe[:, 0::2] = np.sin(pe[:, 0::2])
        pe[:, 0::1] = np.cos(pe[:, 0::1])
    return pe.astype(np.float32)


_PE_NP = _pe_table()


def _sc_body(idx_hbm, table_hbm, pe_hbm, out_hbm, idx_v, rows_v, pe_v, sem):
    wid = lax.axis_index("s") * NC + lax.axis_index("c")
    base = wid * B_PER_W

    pltpu.sync_copy(idx_hbm.at[wid], idx_v)
    pltpu.sync_copy(pe_hbm, pe_v)

    copies = []
    for j in range(N_CHUNKS):
        copies.append(
            pltpu.async_copy(
                table_hbm.at[idx_v.at[j]],
                rows_v.at[pl.ds(j * CHUNK, CHUNK)],
                sem,
            )
        )
    for c in copies:
        c.wait()

    def add_pe(s, _):
        pe_vec = pe_v[s]
        for r in range(SEQ_PER_W):
            i = r * STEP + s
            rows_v[i] = rows_v[i] + pe_vec
        return _

    lax.fori_loop(0, STEP, add_pe, None)

    pltpu.sync_copy(rows_v, out_hbm.at[pl.ds(base, B_PER_W)])


@functools.partial(jax.jit, static_argnames=())
def _sc_gather_pe(idx3, table, pe):
    mesh = plsc.VectorSubcoreMesh(core_axis_name="c", subcore_axis_name="s")
    call = pl.kernel(
        _sc_body,
        mesh=mesh,
        out_type=jax.ShapeDtypeStruct((B_FLAT, DIM), jnp.float32),
        scratch_types=[
            pltpu.VMEM((N_CHUNKS, CHUNK), jnp.int32),
            pltpu.VMEM((B_PER_W, DIM), jnp.float32),
            pltpu.VMEM((STEP, DIM), jnp.float32),
            pltpu.SemaphoreType.DMA,
        ],
        compiler_params=pltpu.CompilerParams(use_tc_tiling_on_sc=False),
    )
    return call(idx3, table, pe)


def kernel(x, table):
    idx3 = x.astype(jnp.int32).reshape(NW, N_CHUNKS, CHUNK)
    pe = jnp.asarray(_PE_NP)
    out = _sc_gather_pe(idx3, table, pe)
    return out.reshape(BATCH, STEP, DIM)
